# baseline (speedup 1.0000x reference)
import functools

import jax
import jax.numpy as jnp
from jax.experimental import pallas as pl
from jax.experimental.pallas import tpu as pltpu

_C = 256
_HID = 1024


def _mlp_block(x_ref, w1, b1_ref, w2, b2_ref, o_ref):
    x = x_ref[...].astype(jnp.bfloat16)
    h = jax.lax.dot_general(
        x, w1, (((1,), (0,)), ((), ())),
        preferred_element_type=jnp.float32,
    )
    h = jnp.maximum(h + b1_ref[...], 0.0).astype(jnp.bfloat16)
    o = jax.lax.dot_general(
        h, w2, (((1,), (0,)), ((), ())),
        preferred_element_type=jnp.float32,
    )
    o_ref[...] = o + b2_ref[...]


def _make_kernel(nblk0):
    def _kernel(x0_ref, x1_ref, w1_ref, b1_ref, w2_ref, b2_ref,
                o0_ref, o1_ref):
        j = pl.program_id(0)
        w1 = w1_ref[...].astype(jnp.bfloat16)
        w2 = w2_ref[...].astype(jnp.bfloat16)

        @pl.when(j < nblk0)
        def _():
            _mlp_block(x0_ref, w1, b1_ref, w2, b2_ref, o0_ref)

        @pl.when(j >= nblk0)
        def _():
            _mlp_block(x1_ref, w1, b1_ref, w2, b2_ref, o1_ref)

    return _kernel


@functools.partial(jax.jit, static_argnames=("block_t", "interpret"))
def _mlp_two(x0, x1, w1, b1r, w2, b2r, *, block_t, interpret=False):
    n0, c = x0.shape
    n1, _ = x1.shape
    bt = block_t
    nblk0, nblk1 = n0 // bt, n1 // bt
    grid = (nblk0 + nblk1,)
    return pl.pallas_call(
        _make_kernel(nblk0),
        grid=grid,
        in_specs=[
            pl.BlockSpec((bt, c), lambda j: (jnp.minimum(j, nblk0 - 1), 0)),
            pl.BlockSpec((bt, c), lambda j: (jnp.maximum(j - nblk0, 0), 0)),
            pl.BlockSpec((_C, _HID), lambda j: (0, 0)),
            pl.BlockSpec((1, _HID), lambda j: (0, 0)),
            pl.BlockSpec((_HID, _C), lambda j: (0, 0)),
            pl.BlockSpec((1, _C), lambda j: (0, 0)),
        ],
        out_specs=[
            pl.BlockSpec((bt, c), lambda j: (jnp.minimum(j, nblk0 - 1), 0)),
            pl.BlockSpec((bt, c), lambda j: (jnp.maximum(j - nblk0, 0), 0)),
        ],
        out_shape=[
            jax.ShapeDtypeStruct((n0, c), jnp.float32),
            jax.ShapeDtypeStruct((n1, c), jnp.float32),
        ],
        compiler_params=pltpu.CompilerParams(
            dimension_semantics=("parallel",),
        ),
        interpret=interpret,
    )(x0, x1, w1, b1r, w2, b2r)


def kernel(feat_map0, feat_map1, feat_map2, feat_map3, W1, b1, W2, b2):
    b, c, h0, w0 = feat_map0.shape
    _, _, h1, w1sz = feat_map1.shape
    xt0 = feat_map0.transpose(0, 2, 3, 1).reshape(-1, c)
    xt1 = feat_map1.transpose(0, 2, 3, 1).reshape(-1, c)
    y0, y1 = _mlp_two(xt0, xt1, W1, b1.reshape(1, _HID),
                      W2, b2.reshape(1, _C), block_t=1024)
    out0 = y0.reshape(b, h0, w0, c).transpose(0, 3, 1, 2)
    out1 = y1.reshape(b, h1, w1sz, c).transpose(0, 3, 1, 2)
    return (out0, out1, feat_map2, feat_map3)


# speedup vs baseline: 1.1268x; 1.1268x over previous
import functools

import jax
import jax.numpy as jnp
from jax.experimental import pallas as pl
from jax.experimental.pallas import tpu as pltpu

_C = 256
_HID = 1024


def _mlp_block(x_ref, w1, b1_ref, w2, b2_ref, o_ref):
    x = x_ref[...].astype(jnp.bfloat16)
    h = jax.lax.dot_general(
        x, w1, (((1,), (0,)), ((), ())),
        preferred_element_type=jnp.float32,
    )
    h = jnp.maximum(h + b1_ref[...], 0.0).astype(jnp.bfloat16)
    o = jax.lax.dot_general(
        h, w2, (((1,), (0,)), ((), ())),
        preferred_element_type=jnp.float32,
    )
    o_ref[...] = o + b2_ref[...]


def _make_kernel(nblk0):
    def _kernel(x0_ref, x1_ref, w1_ref, b1_ref, w2_ref, b2_ref,
                o0_ref, o1_ref):
        j = pl.program_id(0)
        w1 = w1_ref[...].astype(jnp.bfloat16)
        w2 = w2_ref[...].astype(jnp.bfloat16)

        @pl.when(j < nblk0)
        def _():
            _mlp_block(x0_ref, w1, b1_ref, w2, b2_ref, o0_ref)

        @pl.when(j >= nblk0)
        def _():
            _mlp_block(x1_ref, w1, b1_ref, w2, b2_ref, o1_ref)

    return _kernel


@functools.partial(jax.jit, static_argnames=("block_t", "interpret"))
def _mlp_two(x0, x1, w1, b1r, w2, b2r, *, block_t, interpret=False):
    n0, c = x0.shape
    n1, _ = x1.shape
    bt = block_t
    nblk0, nblk1 = n0 // bt, n1 // bt
    grid = (nblk0 + nblk1,)
    return pl.pallas_call(
        _make_kernel(nblk0),
        grid=grid,
        in_specs=[
            pl.BlockSpec((bt, c), lambda j: (jnp.minimum(j, nblk0 - 1), 0)),
            pl.BlockSpec((bt, c), lambda j: (jnp.maximum(j - nblk0, 0), 0)),
            pl.BlockSpec((_C, _HID), lambda j: (0, 0)),
            pl.BlockSpec((1, _HID), lambda j: (0, 0)),
            pl.BlockSpec((_HID, _C), lambda j: (0, 0)),
            pl.BlockSpec((1, _C), lambda j: (0, 0)),
        ],
        out_specs=[
            pl.BlockSpec((bt, c), lambda j: (jnp.minimum(j, nblk0 - 1), 0)),
            pl.BlockSpec((bt, c), lambda j: (jnp.maximum(j - nblk0, 0), 0)),
        ],
        out_shape=[
            jax.ShapeDtypeStruct((n0, c), jnp.float32),
            jax.ShapeDtypeStruct((n1, c), jnp.float32),
        ],
        compiler_params=pltpu.CompilerParams(
            dimension_semantics=("parallel",),
        ),
        interpret=interpret,
    )(x0, x1, w1, b1r, w2, b2r)


def kernel(feat_map0, feat_map1, feat_map2, feat_map3, W1, b1, W2, b2):
    b, c, h0, w0 = feat_map0.shape
    _, _, h1, w1sz = feat_map1.shape
    xt0 = feat_map0.transpose(0, 2, 3, 1).reshape(-1, c)
    xt1 = feat_map1.transpose(0, 2, 3, 1).reshape(-1, c)
    y0, y1 = _mlp_two(xt0, xt1, W1, b1.reshape(1, _HID),
                      W2, b2.reshape(1, _C), block_t=4096)
    out0 = y0.reshape(b, h0, w0, c).transpose(0, 3, 1, 2)
    out1 = y1.reshape(b, h1, w1sz, c).transpose(0, 3, 1, 2)
    return (out0, out1, feat_map2, feat_map3)


# speedup vs baseline: 1.1537x; 1.0238x over previous
import functools

import jax
import jax.numpy as jnp
from jax.experimental import pallas as pl
from jax.experimental.pallas import tpu as pltpu

_C = 256
_HID = 1024


def _mlp_block(x_ref, w1, b1_ref, w2, b2_ref, o_ref):
    x = x_ref[...].astype(jnp.bfloat16)
    h = jax.lax.dot_general(
        x, w1, (((1,), (0,)), ((), ())),
        preferred_element_type=jnp.float32,
    )
    h = jnp.maximum(h + b1_ref[...], 0.0).astype(jnp.bfloat16)
    o = jax.lax.dot_general(
        h, w2, (((1,), (0,)), ((), ())),
        preferred_element_type=jnp.float32,
    )
    o_ref[...] = o + b2_ref[...]


def _make_kernel(nblk0):
    def _kernel(x0_ref, x1_ref, p2_ref, p3_ref, w1_ref, b1_ref, w2_ref,
                b2_ref, o0_ref, o1_ref, q2_ref, q3_ref):
        j = pl.program_id(0)
        w1 = w1_ref[...].astype(jnp.bfloat16)
        w2 = w2_ref[...].astype(jnp.bfloat16)

        @pl.when(j < nblk0)
        def _():
            _mlp_block(x0_ref, w1, b1_ref, w2, b2_ref, o0_ref)

        @pl.when(j >= nblk0)
        def _():
            _mlp_block(x1_ref, w1, b1_ref, w2, b2_ref, o1_ref)

        @pl.when(j == 0)
        def _():
            q2_ref[...] = p2_ref[...]
            q3_ref[...] = p3_ref[...]

    return _kernel


@functools.partial(jax.jit, static_argnames=("block_t", "interpret"))
def _sparse3d_call(x0, x1, p2, p3, w1, b1r, w2, b2r, *, block_t,
                   interpret=False):
    n0, c = x0.shape
    n1, _ = x1.shape
    bt = block_t
    nblk0, nblk1 = n0 // bt, n1 // bt
    grid = (nblk0 + nblk1,)
    const = lambda j: (0, 0)
    return pl.pallas_call(
        _make_kernel(nblk0),
        grid=grid,
        in_specs=[
            pl.BlockSpec((bt, c), lambda j: (jnp.minimum(j, nblk0 - 1), 0)),
            pl.BlockSpec((bt, c), lambda j: (jnp.maximum(j - nblk0, 0), 0)),
            pl.BlockSpec(p2.shape, const),
            pl.BlockSpec(p3.shape, const),
            pl.BlockSpec((_C, _HID), const),
            pl.BlockSpec((1, _HID), const),
            pl.BlockSpec((_HID, _C), const),
            pl.BlockSpec((1, _C), const),
        ],
        out_specs=[
            pl.BlockSpec((bt, c), lambda j: (jnp.minimum(j, nblk0 - 1), 0)),
            pl.BlockSpec((bt, c), lambda j: (jnp.maximum(j - nblk0, 0), 0)),
            pl.BlockSpec(p2.shape, const),
            pl.BlockSpec(p3.shape, const),
        ],
        out_shape=[
            jax.ShapeDtypeStruct((n0, c), jnp.float32),
            jax.ShapeDtypeStruct((n1, c), jnp.float32),
            jax.ShapeDtypeStruct(p2.shape, jnp.float32),
            jax.ShapeDtypeStruct(p3.shape, jnp.float32),
        ],
        compiler_params=pltpu.CompilerParams(
            dimension_semantics=("arbitrary",),
            vmem_limit_bytes=100 * 1024 * 1024,
        ),
        interpret=interpret,
    )(x0, x1, p2, p3, w1, b1r, w2, b2r)


def _tokens(feat):
    b, c, h, w = feat.shape
    return feat.transpose(0, 2, 3, 1).reshape(-1, c)


def _from_tokens(y, shape):
    b, c, h, w = shape
    return y.reshape(b, h, w, c).transpose(0, 3, 1, 2)


def kernel(feat_map0, feat_map1, feat_map2, feat_map3, W1, b1, W2, b2):
    y0, y1, q2, q3 = _sparse3d_call(
        _tokens(feat_map0), _tokens(feat_map1),
        _tokens(feat_map2), _tokens(feat_map3),
        W1, b1.reshape(1, _HID), W2, b2.reshape(1, _C), block_t=2048)
    return (_from_tokens(y0, feat_map0.shape),
            _from_tokens(y1, feat_map1.shape),
            _from_tokens(q2, feat_map2.shape),
            _from_tokens(q3, feat_map3.shape))
